# baseline (speedup 1.0000x reference)
import jax
import jax.numpy as jnp
from jax import lax
from jax.experimental import pallas as pl
from jax.experimental.pallas import tpu as pltpu
from jax.experimental.pallas import tpu_sc as plsc

BATCH = 4096
HIST = 200
EMB = 128
NB = BATCH * HIST
NC = 2
NS = 16
NW = NC * NS
ROWS_PER_W = NB // NW
CHUNK = 128
P = 4
SPAN = ROWS_PER_W // 2
NGROUP = SPAN // CHUNK


def _pipelined_gather(idx_hbm, table_sh, out_hbm, base, idx_all, rowsb,
                      gsems, wsems):
  pltpu.sync_copy(idx_hbm.at[pl.ds(base, SPAN)], idx_all)

  def idxref(g):
    return idx_all.at[pl.ds(g * CHUNK, CHUNK)]

  def gstart(g, b):
    pltpu.async_copy(table_sh.at[idxref(g)], rowsb[b], gsems[b])

  def gwait(g, b):
    pltpu.make_async_copy(table_sh.at[idxref(g)], rowsb[b], gsems[b]).wait()

  def wstart(g, b):
    start = base + g * CHUNK
    pltpu.async_copy(rowsb[b], out_hbm.at[pl.ds(start, CHUNK)], wsems[b])

  def wwait(b):
    pltpu.make_async_copy(rowsb[b], out_hbm.at[pl.ds(0, CHUNK)],
                          wsems[b]).wait()

  gstart(0, 0)
  for g in range(P - 1):
    gstart(g + 1, (g + 1) % P)
    gwait(g, g % P)
    wstart(g, g % P)

  def step(g, b):
    wwait((b + 1) % P)
    gstart(g + 1, (b + 1) % P)
    gwait(g, b)
    wstart(g, b)

  n_uniform = NGROUP - P
  def fori_body(s, _):
    g0 = P * s + (P - 1)
    for j in range(P):
      step(g0 + j, (P - 1 + j) % P)
    return _

  lax.fori_loop(0, n_uniform // P, fori_body, 0)
  for g in range(P - 1 + P * (n_uniform // P), NGROUP - 1):
    step(g, g % P)

  bl = (NGROUP - 1) % P
  wwait((bl + 1) % P)
  gwait(NGROUP - 1, bl)
  wstart(NGROUP - 1, bl)
  for j in range(P - 1):
    wwait((bl + 2 + j) % P)


def _embed_body(dx_idx_hbm, proc_idx_hbm, wdx_hbm, wproc_hbm,
                out_dx_hbm, out_proc_hbm,
                idx_all,
                rows0, rows1, rows2, rows3,
                sh_dx, sh_proc,
                gsem0, gsem1, gsem2, gsem3,
                wsem0, wsem1, wsem2, wsem3):
  sid = lax.axis_index("s")
  wid = sid * NC + lax.axis_index("c")
  base = wid * ROWS_PER_W
  @pl.when(sid == 0)
  def _stage():
    pltpu.sync_copy(wdx_hbm, sh_dx)
    pltpu.sync_copy(wproc_hbm, sh_proc)
  plsc.subcore_barrier()
  rowsb = (rows0, rows1, rows2, rows3)
  gsems = (gsem0, gsem1, gsem2, gsem3)
  wsems = (wsem0, wsem1, wsem2, wsem3)
  def halves_body(half, _):
    _pipelined_gather(dx_idx_hbm, sh_dx, out_dx_hbm, base + half * SPAN,
                      idx_all, rowsb, gsems, wsems)
    _pipelined_gather(proc_idx_hbm, sh_proc, out_proc_hbm, base + half * SPAN,
                      idx_all, rowsb, gsems, wsems)
    return _

  lax.fori_loop(0, 2, halves_body, 0)


@jax.jit
def _embed(dx_flat, proc_flat, W_dx, W_proc):
  mesh = plsc.VectorSubcoreMesh(core_axis_name="c", subcore_axis_name="s")
  return pl.kernel(
      _embed_body,
      out_type=(
          jax.ShapeDtypeStruct((NB, EMB), jnp.float32),
          jax.ShapeDtypeStruct((NB, EMB), jnp.float32),
      ),
      mesh=mesh,
      scratch_types=[
          pltpu.VMEM((SPAN,), jnp.int32),
          pltpu.VMEM((CHUNK, EMB), jnp.float32),
          pltpu.VMEM((CHUNK, EMB), jnp.float32),
          pltpu.VMEM((CHUNK, EMB), jnp.float32),
          pltpu.VMEM((CHUNK, EMB), jnp.float32),
          pltpu.VMEM_SHARED(W_dx.shape, jnp.float32),
          pltpu.VMEM_SHARED(W_proc.shape, jnp.float32),
          pltpu.SemaphoreType.DMA,
          pltpu.SemaphoreType.DMA,
          pltpu.SemaphoreType.DMA,
          pltpu.SemaphoreType.DMA,
          pltpu.SemaphoreType.DMA,
          pltpu.SemaphoreType.DMA,
          pltpu.SemaphoreType.DMA,
          pltpu.SemaphoreType.DMA,
      ],
  )(dx_flat, proc_flat, W_dx, W_proc)


def kernel(dx_ints, proc_ints, W_dx, W_proc, visit, max_num_codes):
  dx_flat = dx_ints.reshape(NB)
  proc_flat = proc_ints.reshape(NB)
  out_dx, out_proc = _embed(dx_flat, proc_flat, W_dx, W_proc)
  emb_dx = out_dx.reshape(BATCH, HIST, EMB)
  emb_proc = out_proc.reshape(BATCH, HIST, EMB)
  mask_dx = jnp.ones((BATCH, HIST, 1), dtype=jnp.float32)
  mask_proc = jnp.ones((BATCH, HIST, 1), dtype=jnp.float32)
  emb_visit = jnp.broadcast_to(visit[None, :, :], (1, visit.shape[0], EMB))
  mask_visit = jnp.ones((1, 1), dtype=jnp.float32)
  return (emb_dx, emb_proc, emb_visit, mask_dx, mask_proc, mask_visit)


# speedup vs baseline: 1.0049x; 1.0049x over previous
import jax
import jax.numpy as jnp
from jax import lax
from jax.experimental import pallas as pl
from jax.experimental.pallas import tpu as pltpu
from jax.experimental.pallas import tpu_sc as plsc

BATCH = 4096
HIST = 200
EMB = 128
NB = BATCH * HIST
NC = 2
NS = 16
NW = NC * NS
ROWS_PER_W = NB // NW
CHUNK = 128
P = 4
SPAN = ROWS_PER_W // 2
NGROUP = SPAN // CHUNK


def _pipelined_gather(idx_hbm, table_sh, out_hbm, base, idx_all, rowsb,
                      gsems, wsems, first, last):
  pltpu.sync_copy(idx_hbm.at[pl.ds(base, SPAN)], idx_all)

  def idxref(g):
    return idx_all.at[pl.ds(g * CHUNK, CHUNK)]

  def gstart(g, b):
    pltpu.async_copy(table_sh.at[idxref(g)], rowsb[b], gsems[b])

  def gwait(g, b):
    pltpu.make_async_copy(table_sh.at[idxref(g)], rowsb[b], gsems[b]).wait()

  def wstart(g, b):
    start = base + g * CHUNK
    pltpu.async_copy(rowsb[b], out_hbm.at[pl.ds(start, CHUNK)], wsems[b])

  def wwait(b):
    pltpu.make_async_copy(rowsb[b], out_hbm.at[pl.ds(0, CHUNK)],
                          wsems[b]).wait()

  gstart(0, 0)
  for g in range(P - 1):
    if not first:
      wwait((g + 1) % P)
    gstart(g + 1, (g + 1) % P)
    gwait(g, g % P)
    wstart(g, g % P)

  def step(g, b):
    wwait((b + 1) % P)
    gstart(g + 1, (b + 1) % P)
    gwait(g, b)
    wstart(g, b)

  n_uniform = NGROUP - P
  def fori_body(s, _):
    g0 = P * s + (P - 1)
    for j in range(P):
      step(g0 + j, (P - 1 + j) % P)
    return _

  lax.fori_loop(0, n_uniform // P, fori_body, 0)
  for g in range(P - 1 + P * (n_uniform // P), NGROUP - 1):
    step(g, g % P)

  bl = (NGROUP - 1) % P
  wwait((bl + 1) % P)
  gwait(NGROUP - 1, bl)
  wstart(NGROUP - 1, bl)
  if last:
    for j in range(P - 1):
      wwait((bl + 2 + j) % P)


def _embed_body(dx_idx_hbm, proc_idx_hbm, wdx_hbm, wproc_hbm,
                out_dx_hbm, out_proc_hbm,
                idx_all,
                rows0, rows1, rows2, rows3,
                sh_dx, sh_proc,
                gsem0, gsem1, gsem2, gsem3,
                wsem0, wsem1, wsem2, wsem3):
  sid = lax.axis_index("s")
  wid = sid * NC + lax.axis_index("c")
  base = wid * ROWS_PER_W
  @pl.when(sid == 0)
  def _stage():
    pltpu.sync_copy(wdx_hbm, sh_dx)
    pltpu.sync_copy(wproc_hbm, sh_proc)
  plsc.subcore_barrier()
  rowsb = (rows0, rows1, rows2, rows3)
  gsems = (gsem0, gsem1, gsem2, gsem3)
  wsems = (wsem0, wsem1, wsem2, wsem3)
  phases = [(dx_idx_hbm, sh_dx, out_dx_hbm, 0),
            (proc_idx_hbm, sh_proc, out_proc_hbm, 0),
            (dx_idx_hbm, sh_dx, out_dx_hbm, SPAN),
            (proc_idx_hbm, sh_proc, out_proc_hbm, SPAN)]
  for i, (idx_hbm, table_sh, out_hbm, off) in enumerate(phases):
    _pipelined_gather(idx_hbm, table_sh, out_hbm, base + off,
                      idx_all, rowsb, gsems, wsems,
                      first=(i == 0), last=(i == len(phases) - 1))


@jax.jit
def _embed(dx_flat, proc_flat, W_dx, W_proc):
  mesh = plsc.VectorSubcoreMesh(core_axis_name="c", subcore_axis_name="s")
  return pl.kernel(
      _embed_body,
      out_type=(
          jax.ShapeDtypeStruct((NB, EMB), jnp.float32),
          jax.ShapeDtypeStruct((NB, EMB), jnp.float32),
      ),
      mesh=mesh,
      scratch_types=[
          pltpu.VMEM((SPAN,), jnp.int32),
          pltpu.VMEM((CHUNK, EMB), jnp.float32),
          pltpu.VMEM((CHUNK, EMB), jnp.float32),
          pltpu.VMEM((CHUNK, EMB), jnp.float32),
          pltpu.VMEM((CHUNK, EMB), jnp.float32),
          pltpu.VMEM_SHARED(W_dx.shape, jnp.float32),
          pltpu.VMEM_SHARED(W_proc.shape, jnp.float32),
          pltpu.SemaphoreType.DMA,
          pltpu.SemaphoreType.DMA,
          pltpu.SemaphoreType.DMA,
          pltpu.SemaphoreType.DMA,
          pltpu.SemaphoreType.DMA,
          pltpu.SemaphoreType.DMA,
          pltpu.SemaphoreType.DMA,
          pltpu.SemaphoreType.DMA,
      ],
  )(dx_flat, proc_flat, W_dx, W_proc)


def kernel(dx_ints, proc_ints, W_dx, W_proc, visit, max_num_codes):
  dx_flat = dx_ints.reshape(NB)
  proc_flat = proc_ints.reshape(NB)
  out_dx, out_proc = _embed(dx_flat, proc_flat, W_dx, W_proc)
  emb_dx = out_dx.reshape(BATCH, HIST, EMB)
  emb_proc = out_proc.reshape(BATCH, HIST, EMB)
  mask_dx = jnp.ones((BATCH, HIST, 1), dtype=jnp.float32)
  mask_proc = jnp.ones((BATCH, HIST, 1), dtype=jnp.float32)
  emb_visit = jnp.broadcast_to(visit[None, :, :], (1, visit.shape[0], EMB))
  mask_visit = jnp.ones((1, 1), dtype=jnp.float32)
  return (emb_dx, emb_proc, emb_visit, mask_dx, mask_proc, mask_visit)
